# Initial kernel scaffold; baseline (speedup 1.0000x reference)
import functools

import jax
import jax.numpy as jnp
from jax import lax
from jax.experimental import pallas as pl
from jax.experimental.pallas import tpu as pltpu
from jax.experimental.pallas import tpu_sc as plsc

N = 10000
E = 320000

N_PAD = 10240
BN = 640
G = N_PAD // BN
NSUB = 16
K = 128
EPS = E // NSUB
C = (EPS + K - 1) // K
EP = C * K
RPT = N_PAD // NSUB

_mesh = plsc.VectorSubcoreMesh(core_axis_name="c", subcore_axis_name="s")
F32 = jnp.float32


@functools.partial(
    pl.kernel,
    out_type=jax.ShapeDtypeStruct((N_PAD,), F32),
    mesh=_mesh,
    scratch_types=[
        pltpu.VMEM((C, K), jnp.int32),
        pltpu.VMEM((K,), F32),
        pltpu.VMEM_SHARED((N_PAD,), F32),
    ],
)
def _deg_kernel(cidx_hbm, zeros_hbm, deg_hbm, cidx_v, ones_v, acc):
    c = lax.axis_index("c")
    s = lax.axis_index("s")

    @pl.when(c == 0)
    def _():
        base = s * RPT
        pltpu.sync_copy(cidx_hbm.at[s], cidx_v)
        for k in range(K // 16):
            ones_v[pl.ds(k * 16, 16)] = jnp.ones((16,), F32)
        pltpu.sync_copy(zeros_hbm.at[pl.ds(base, RPT)],
                        acc.at[pl.ds(base, RPT)])
        plsc.subcore_barrier()

        def body(j, carry):
            pltpu.sync_copy(ones_v, acc.at[cidx_v.at[j]], add=True)
            return carry

        lax.fori_loop(0, C, body, 0)
        plsc.subcore_barrier()
        pltpu.sync_copy(acc.at[pl.ds(base, RPT)],
                        deg_hbm.at[pl.ds(base, RPT)])


def _make_prop(W):
    @functools.partial(
        pl.kernel,
        out_type=jax.ShapeDtypeStruct((2 * N_PAD, W), F32),
        mesh=_mesh,
        scratch_types=[
            pltpu.VMEM((C, K), jnp.int32),
            pltpu.VMEM((C, K), jnp.int32),
            pltpu.VMEM((K, W), F32),
            pltpu.VMEM_SHARED((N_PAD, W), F32),
            pltpu.SemaphoreType.DMA,
        ],
    )
    def prop(y_hbm, ridx_hbm, cidx_hbm, out_hbm, ridx_v, cidx_v, buf, acc, sem):
        c = lax.axis_index("c")
        s = lax.axis_index("s")
        w32 = c * NSUB + s
        pltpu.sync_copy(ridx_hbm.at[w32], ridx_v)
        pltpu.sync_copy(cidx_hbm.at[s], cidx_v)
        base = s * RPT
        pltpu.sync_copy(y_hbm.at[pl.ds(c * N_PAD + base, RPT)],
                        acc.at[pl.ds(base, RPT)])
        plsc.subcore_barrier()

        def body(j, carry):
            pltpu.async_copy(y_hbm.at[ridx_v.at[j]], buf, sem).wait()
            pltpu.sync_copy(buf, acc.at[cidx_v.at[j]], add=True)
            return carry

        lax.fori_loop(0, C, body, 0)
        plsc.subcore_barrier()
        pltpu.sync_copy(acc.at[pl.ds(base, RPT)],
                        out_hbm.at[pl.ds(c * N_PAD + base, RPT)])

    return prop


_prop64 = _make_prop(64)
_prop192 = _make_prop(192)


def _dot(a, b):
    return jax.lax.dot_general(a, b, (((1,), (0,)), ((), ())),
                               preferred_element_type=F32,
                               precision=jax.lax.Precision.HIGHEST)


def _t0_body(deg_ref, x_ref, w_ref, b_ref, u_ref, y_ref, s_ref):
    c = pl.program_id(1)
    sv = jax.lax.rsqrt(deg_ref[...] + 1.0)

    @pl.when(c == 0)
    def _():
        s_ref[...] = sv
        u_ref[...] = _dot(x_ref[...], w_ref[...]) + b_ref[...]
        y_ref[...] = x_ref[:, :64] * sv

    @pl.when(c == 1)
    def _():
        y_ref[...] = x_ref[:, 64:] * sv


def _t0(deg2, x_p, w00, b00):
    return pl.pallas_call(
        _t0_body,
        grid=(G, 2),
        in_specs=[
            pl.BlockSpec((BN, 1), lambda i, c: (i, 0)),
            pl.BlockSpec((BN, 128), lambda i, c: (i, 0)),
            pl.BlockSpec((128, 128), lambda i, c: (0, 0)),
            pl.BlockSpec((1, 128), lambda i, c: (0, 0)),
        ],
        out_specs=[
            pl.BlockSpec((BN, 128), lambda i, c: (i, 0)),
            pl.BlockSpec((BN, 64), lambda i, c: (c * G + i, 0)),
            pl.BlockSpec((BN, 1), lambda i, c: (i, 0)),
        ],
        out_shape=[
            jax.ShapeDtypeStruct((N_PAD, 128), F32),
            jax.ShapeDtypeStruct((2 * N_PAD, 64), F32),
            jax.ShapeDtypeStruct((N_PAD, 1), F32),
        ],
    )(deg2, x_p, w00, b00)


def _make_t1(W):
    def body(tlo_ref, thi_ref, s_ref, w_ref, b_ref, u_ref, y_ref):
        c = pl.program_id(1)
        sv = s_ref[...]

        @pl.when(c == 0)
        def _():
            h = jnp.concatenate([tlo_ref[...] * sv, thi_ref[...] * sv], axis=1)
            u_ref[...] = _dot(h, w_ref[...]) + b_ref[...]
            y_ref[...] = tlo_ref[...] * (sv * sv)

        @pl.when(c == 1)
        def _():
            y_ref[...] = thi_ref[...] * (sv * sv)

    D = 2 * W

    def run(ts, s2, wm, b2):
        return pl.pallas_call(
            body,
            grid=(G, 2),
            in_specs=[
                pl.BlockSpec((BN, W), lambda i, c: (i, 0)),
                pl.BlockSpec((BN, W), lambda i, c: (G + i, 0)),
                pl.BlockSpec((BN, 1), lambda i, c: (i, 0)),
                pl.BlockSpec((D, 128), lambda i, c: (0, 0)),
                pl.BlockSpec((1, 128), lambda i, c: (0, 0)),
            ],
            out_specs=[
                pl.BlockSpec((BN, 128), lambda i, c: (i, 0)),
                pl.BlockSpec((BN, W), lambda i, c: (c * G + i, 0)),
            ],
            out_shape=[
                jax.ShapeDtypeStruct((N_PAD, 128), F32),
                jax.ShapeDtypeStruct((2 * N_PAD, W), F32),
            ],
        )(ts, s2, wm, b2)

    return run


_t1_64 = _make_t1(64)
_t1_192 = _make_t1(192)


def _t2a_body(tlo_ref, thi_ref, s_ref, u0_ref, u1_ref, w2_ref, b2_ref,
              w0n_ref, b0n_ref, u0p_ref, y_ref, h_ref):
    c = pl.program_id(1)
    sv = s_ref[...]

    @pl.when(c == 0)
    def _():
        h2 = jnp.concatenate([tlo_ref[...] * sv, thi_ref[...] * sv], axis=1)
        u2 = _dot(h2, w2_ref[...]) + b2_ref[...]
        hl = jax.nn.relu(jnp.concatenate([u0_ref[...], u1_ref[...], u2], axis=1))
        h_ref[...] = hl
        u0p_ref[...] = _dot(hl, w0n_ref[...]) + b0n_ref[...]
        y_ref[...] = hl[:, :192] * sv

    @pl.when(c == 1)
    def _():
        y_ref[...] = h_ref[:, 192:] * sv


def _t2a(t2s, s2, u0, u1, w02, b02, w10, b10):
    return pl.pallas_call(
        _t2a_body,
        grid=(G, 2),
        in_specs=[
            pl.BlockSpec((BN, 64), lambda i, c: (i, 0)),
            pl.BlockSpec((BN, 64), lambda i, c: (G + i, 0)),
            pl.BlockSpec((BN, 1), lambda i, c: (i, 0)),
            pl.BlockSpec((BN, 128), lambda i, c: (i, 0)),
            pl.BlockSpec((BN, 128), lambda i, c: (i, 0)),
            pl.BlockSpec((128, 128), lambda i, c: (0, 0)),
            pl.BlockSpec((1, 128), lambda i, c: (0, 0)),
            pl.BlockSpec((384, 128), lambda i, c: (0, 0)),
            pl.BlockSpec((1, 128), lambda i, c: (0, 0)),
        ],
        out_specs=[
            pl.BlockSpec((BN, 128), lambda i, c: (i, 0)),
            pl.BlockSpec((BN, 192), lambda i, c: (c * G + i, 0)),
        ],
        out_shape=[
            jax.ShapeDtypeStruct((N_PAD, 128), F32),
            jax.ShapeDtypeStruct((2 * N_PAD, 192), F32),
        ],
        scratch_shapes=[pltpu.VMEM((BN, 384), F32)],
    )(t2s, s2, u0, u1, w02, b02, w10, b10)


def _t2b_body(tlo_ref, thi_ref, s_ref, u0_ref, u1_ref, w2_ref, b2_ref, o_ref):
    sv = s_ref[...]
    h2 = jnp.concatenate([tlo_ref[...] * sv, thi_ref[...] * sv], axis=1)
    u2 = _dot(h2, w2_ref[...]) + b2_ref[...]
    o_ref[...] = jax.nn.relu(jnp.concatenate([u0_ref[...], u1_ref[...], u2], axis=1))


def _t2b(t2s, s2, u0p, u1p, w12, b12):
    return pl.pallas_call(
        _t2b_body,
        grid=(G,),
        in_specs=[
            pl.BlockSpec((BN, 192), lambda i: (i, 0)),
            pl.BlockSpec((BN, 192), lambda i: (G + i, 0)),
            pl.BlockSpec((BN, 1), lambda i: (i, 0)),
            pl.BlockSpec((BN, 128), lambda i: (i, 0)),
            pl.BlockSpec((BN, 128), lambda i: (i, 0)),
            pl.BlockSpec((384, 128), lambda i: (0, 0)),
            pl.BlockSpec((1, 128), lambda i: (0, 0)),
        ],
        out_specs=pl.BlockSpec((BN, 384), lambda i: (i, 0)),
        out_shape=jax.ShapeDtypeStruct((N_PAD, 384), F32),
    )(t2s, s2, u0p, u1p, w12, b12)


def kernel(x, edge_index, W0_0, b0_0, W0_1, b0_1, W0_2, b0_2,
           W1_0, b1_0, W1_1, b1_1, W1_2, b1_2):
    row = edge_index[0]
    col = edge_index[1]
    colp = jnp.full((NSUB, EP), N, jnp.int32).at[:, :EPS].set(
        col.reshape(NSUB, EPS))
    cidx = colp.reshape(NSUB, C, K)
    rowp = jnp.zeros((NSUB, EP), jnp.int32).at[:, :EPS].set(
        row.reshape(NSUB, EPS))
    ridx = jnp.concatenate([rowp, rowp + N_PAD], axis=0).reshape(2 * NSUB, C, K)

    x_p = jnp.zeros((N_PAD, 128), F32).at[:N].set(x)
    zeros_np = jnp.zeros((N_PAD,), F32)
    b00 = b0_0.reshape(1, 128)
    b01 = b0_1.reshape(1, 128)
    b02 = b0_2.reshape(1, 128)
    b10 = b1_0.reshape(1, 128)
    b11 = b1_1.reshape(1, 128)
    b12 = b1_2.reshape(1, 128)

    deg = _deg_kernel(cidx, zeros_np)
    deg2 = deg.reshape(N_PAD, 1)

    u0, y0s, s2 = _t0(deg2, x_p, W0_0, b00)
    t1s = _prop64(y0s, ridx, cidx)
    u1, y1s = _t1_64(t1s, s2, W0_1, b01)
    t2s = _prop64(y1s, ridx, cidx)

    u0p, ya = _t2a(t2s, s2, u0, u1, W0_2, b02, W1_0, b10)

    ta = _prop192(ya, ridx, cidx)
    u1p, yb = _t1_192(ta, s2, W1_1, b11)
    tb = _prop192(yb, ridx, cidx)
    out = _t2b(tb, s2, u0p, u1p, W1_2, b12)

    return out[:N]


# speedup vs baseline: 7.1788x; 7.1788x over previous
import functools

import jax
import jax.numpy as jnp
from jax import lax
from jax.experimental import pallas as pl
from jax.experimental.pallas import tpu as pltpu
from jax.experimental.pallas import tpu_sc as plsc

N = 10000
E = 320000

N_PAD = 10240
BN = 640
G = N_PAD // BN
NSUB = 16
NW = 32
K = 128
EPS = E // NW
C = (EPS + K - 1) // K
EP = C * K
RPT = N_PAD // NSUB

_mesh = plsc.VectorSubcoreMesh(core_axis_name="c", subcore_axis_name="s")
F32 = jnp.float32


@functools.partial(
    pl.kernel,
    out_type=jax.ShapeDtypeStruct((2, N_PAD), F32),
    mesh=_mesh,
    scratch_types=[
        pltpu.VMEM((C, K), jnp.int32),
        pltpu.VMEM((K,), F32),
        pltpu.VMEM_SHARED((N_PAD,), F32),
    ],
)
def _deg_kernel(cidx_hbm, zeros_hbm, deg_hbm, cidx_v, ones_v, acc):
    c = lax.axis_index("c")
    s = lax.axis_index("s")
    w32 = c * NSUB + s
    base = s * RPT
    pltpu.sync_copy(cidx_hbm.at[w32], cidx_v)
    for k in range(K // 16):
        ones_v[pl.ds(k * 16, 16)] = jnp.ones((16,), F32)
    pltpu.sync_copy(zeros_hbm.at[pl.ds(base, RPT)],
                    acc.at[pl.ds(base, RPT)])
    plsc.subcore_barrier()

    def body(j, carry):
        pltpu.sync_copy(ones_v, acc.at[cidx_v.at[j]], add=True)
        return carry

    lax.fori_loop(0, C, body, 0)
    plsc.subcore_barrier()
    pltpu.sync_copy(acc.at[pl.ds(base, RPT)],
                    deg_hbm.at[c, pl.ds(base, RPT)])


@functools.partial(
    pl.kernel,
    out_type=jax.ShapeDtypeStruct((2, N_PAD, 128), F32),
    mesh=_mesh,
    scratch_types=[
        pltpu.VMEM((C, K), jnp.int32),
        pltpu.VMEM((C, K), jnp.int32),
        pltpu.VMEM((K, 128), F32),
        pltpu.VMEM_SHARED((N_PAD, 128), F32),
        pltpu.SemaphoreType.DMA,
    ],
)
def _prop(y_hbm, z_hbm, ridx_hbm, cidx_hbm, out_hbm, ridx_v, cidx_v, buf,
          acc, sem):
    c = lax.axis_index("c")
    s = lax.axis_index("s")
    w32 = c * NSUB + s
    pltpu.sync_copy(ridx_hbm.at[w32], ridx_v)
    pltpu.sync_copy(cidx_hbm.at[w32], cidx_v)
    base = s * RPT

    @pl.when(c == 0)
    def _():
        pltpu.sync_copy(y_hbm.at[pl.ds(base, RPT)], acc.at[pl.ds(base, RPT)])

    @pl.when(c == 1)
    def _():
        pltpu.sync_copy(z_hbm.at[pl.ds(base, RPT)], acc.at[pl.ds(base, RPT)])

    plsc.subcore_barrier()

    def body(j, carry):
        pltpu.async_copy(y_hbm.at[ridx_v.at[j]], buf, sem).wait()
        pltpu.sync_copy(buf, acc.at[cidx_v.at[j]], add=True)
        return carry

    lax.fori_loop(0, C, body, 0)
    plsc.subcore_barrier()
    pltpu.sync_copy(acc.at[pl.ds(base, RPT)],
                    out_hbm.at[c, pl.ds(base, RPT)])


def _dot(a, b):
    return jax.lax.dot_general(a, b, (((1,), (0,)), ((), ())),
                               preferred_element_type=F32,
                               precision=jax.lax.Precision.HIGHEST)


def _full(shape):
    return pl.BlockSpec(shape, lambda i: tuple(0 for _ in shape))


def _row(w):
    return pl.BlockSpec((BN, w), lambda i: (i, 0))


def _part(j):
    return pl.BlockSpec((1, BN, 128), lambda i, j=j: (j, i, 0))


def _t0_body(d0_ref, d1_ref, x_ref, w_ref, b_ref, u_ref, y_ref, s_ref):
    deg = d0_ref[0] + d1_ref[0] + 1.0
    sv = jax.lax.rsqrt(deg)
    s_ref[...] = sv
    u_ref[...] = _dot(x_ref[...], w_ref[...]) + b_ref[...]
    y_ref[...] = x_ref[...] * sv


def _t0(deg2, x_p, w00, b00):
    return pl.pallas_call(
        _t0_body,
        grid=(G,),
        in_specs=[
            pl.BlockSpec((1, BN, 1), lambda i: (0, i, 0)),
            pl.BlockSpec((1, BN, 1), lambda i: (1, i, 0)),
            _row(128), _full((128, 128)), _full((1, 128)),
        ],
        out_specs=[_row(128), _row(128), _row(1)],
        out_shape=[
            jax.ShapeDtypeStruct((N_PAD, 128), F32),
            jax.ShapeDtypeStruct((N_PAD, 128), F32),
            jax.ShapeDtypeStruct((N_PAD, 1), F32),
        ],
    )(deg2, deg2, x_p, w00, b00)


def _t1l0_body(pa_ref, pb_ref, s_ref, w_ref, b_ref, u_ref, y_ref):
    sv = s_ref[...]
    t = pa_ref[0] + pb_ref[0]
    h = t * sv
    u_ref[...] = _dot(h, w_ref[...]) + b_ref[...]
    y_ref[...] = t * (sv * sv)


def _t1l0(p, s2, wm, b2):
    return pl.pallas_call(
        _t1l0_body,
        grid=(G,),
        in_specs=[_part(0), _part(1), _row(1),
                  _full((128, 128)), _full((1, 128))],
        out_specs=[_row(128), _row(128)],
        out_shape=[
            jax.ShapeDtypeStruct((N_PAD, 128), F32),
            jax.ShapeDtypeStruct((N_PAD, 128), F32),
        ],
    )(p, p, s2, wm, b2)


def _t2a_body(pa_ref, pb_ref, s_ref, u0_ref, u1_ref, w2_ref, b2_ref,
              w0n_ref, b0n_ref, u0p_ref, y0_ref, y1_ref, y2_ref):
    sv = s_ref[...]
    t = pa_ref[0] + pb_ref[0]
    h2 = t * sv
    u2 = _dot(h2, w2_ref[...]) + b2_ref[...]
    hl = jax.nn.relu(jnp.concatenate([u0_ref[...], u1_ref[...], u2], axis=1))
    u0p_ref[...] = _dot(hl, w0n_ref[...]) + b0n_ref[...]
    y0_ref[...] = hl[:, :128] * sv
    y1_ref[...] = hl[:, 128:256] * sv
    y2_ref[...] = hl[:, 256:] * sv


def _t2a(p, s2, u0, u1, w02, b02, w10, b10):
    return pl.pallas_call(
        _t2a_body,
        grid=(G,),
        in_specs=[_part(0), _part(1), _row(1), _row(128), _row(128),
                  _full((128, 128)), _full((1, 128)),
                  _full((384, 128)), _full((1, 128))],
        out_specs=[_row(128), _row(128), _row(128), _row(128)],
        out_shape=[jax.ShapeDtypeStruct((N_PAD, 128), F32)] * 4,
    )(p, p, s2, u0, u1, w02, b02, w10, b10)


def _t1l1_body(pa0, pb0, pa1, pb1, pa2, pb2, s_ref, w_ref, b_ref,
               u_ref, y0_ref, y1_ref, y2_ref):
    sv = s_ref[...]
    t0 = pa0[0] + pb0[0]
    t1 = pa1[0] + pb1[0]
    t2 = pa2[0] + pb2[0]
    h = jnp.concatenate([t0 * sv, t1 * sv, t2 * sv], axis=1)
    u_ref[...] = _dot(h, w_ref[...]) + b_ref[...]
    y0_ref[...] = t0 * (sv * sv)
    y1_ref[...] = t1 * (sv * sv)
    y2_ref[...] = t2 * (sv * sv)


def _t1l1(q0, q1, q2, s2, wm, b2):
    return pl.pallas_call(
        _t1l1_body,
        grid=(G,),
        in_specs=[_part(0), _part(1), _part(0), _part(1), _part(0), _part(1),
                  _row(1), _full((384, 128)), _full((1, 128))],
        out_specs=[_row(128), _row(128), _row(128), _row(128)],
        out_shape=[jax.ShapeDtypeStruct((N_PAD, 128), F32)] * 4,
    )(q0, q0, q1, q1, q2, q2, s2, wm, b2)


def _t2b_body(pa0, pb0, pa1, pb1, pa2, pb2, s_ref, u0_ref, u1_ref,
              w2_ref, b2_ref, o_ref):
    sv = s_ref[...]
    h2 = jnp.concatenate([(pa0[0] + pb0[0]) * sv,
                          (pa1[0] + pb1[0]) * sv,
                          (pa2[0] + pb2[0]) * sv], axis=1)
    u2 = _dot(h2, w2_ref[...]) + b2_ref[...]
    o_ref[...] = jax.nn.relu(
        jnp.concatenate([u0_ref[...], u1_ref[...], u2], axis=1))


def _t2b(q0, q1, q2, s2, u0p, u1p, w12, b12):
    return pl.pallas_call(
        _t2b_body,
        grid=(G,),
        in_specs=[_part(0), _part(1), _part(0), _part(1), _part(0), _part(1),
                  _row(1), _row(128), _row(128),
                  _full((384, 128)), _full((1, 128))],
        out_specs=_row(384),
        out_shape=jax.ShapeDtypeStruct((N_PAD, 384), F32),
    )(q0, q0, q1, q1, q2, q2, s2, u0p, u1p, w12, b12)


def kernel(x, edge_index, W0_0, b0_0, W0_1, b0_1, W0_2, b0_2,
           W1_0, b1_0, W1_1, b1_1, W1_2, b1_2):
    row = edge_index[0]
    col = edge_index[1]
    cidx = jnp.full((NW, EP), N, jnp.int32).at[:, :EPS].set(
        col.reshape(NW, EPS)).reshape(NW, C, K)
    ridx = jnp.zeros((NW, EP), jnp.int32).at[:, :EPS].set(
        row.reshape(NW, EPS)).reshape(NW, C, K)

    x_p = jnp.zeros((N_PAD, 128), F32).at[:N].set(x)
    zeros1 = jnp.zeros((N_PAD,), F32)
    zeros2 = jnp.zeros((N_PAD, 128), F32)
    b00 = b0_0.reshape(1, 128)
    b01 = b0_1.reshape(1, 128)
    b02 = b0_2.reshape(1, 128)
    b10 = b1_0.reshape(1, 128)
    b11 = b1_1.reshape(1, 128)
    b12 = b1_2.reshape(1, 128)

    deg = _deg_kernel(cidx, zeros1)
    deg2 = deg.reshape(2, N_PAD, 1)

    u0, y0, s2 = _t0(deg2, x_p, W0_0, b00)
    p1 = _prop(y0, zeros2, ridx, cidx)
    u1, y1 = _t1l0(p1, s2, W0_1, b01)
    p2 = _prop(y1, zeros2, ridx, cidx)

    u0p, ya0, ya1, ya2 = _t2a(p2, s2, u0, u1, W0_2, b02, W1_0, b10)

    qa0 = _prop(ya0, zeros2, ridx, cidx)
    qa1 = _prop(ya1, zeros2, ridx, cidx)
    qa2 = _prop(ya2, zeros2, ridx, cidx)
    u1p, yb0, yb1, yb2 = _t1l1(qa0, qa1, qa2, s2, W1_1, b11)
    qb0 = _prop(yb0, zeros2, ridx, cidx)
    qb1 = _prop(yb1, zeros2, ridx, cidx)
    qb2 = _prop(yb2, zeros2, ridx, cidx)
    out = _t2b(qb0, qb1, qb2, s2, u0p, u1p, W1_2, b12)

    return out[:N]
